# Initial kernel scaffold; baseline (speedup 1.0000x reference)
import jax
import jax.numpy as jnp
from jax.experimental import pallas as pl


def kernel(nodes_s, nodes_v, senders, receivers, params):
    raise NotImplementedError("write your pallas kernel here")


# speedup vs baseline: 7.9490x; 7.9490x over previous
import functools
import jax
import jax.numpy as jnp
from jax.experimental import pallas as pl
from jax.experimental.pallas import tpu as pltpu

N_NODES = 10000
N_EDGES = 320000
F = 128
NB = 4
RC = 1.0
M0 = 64
M1 = 21
STEPS = 3

BE = 2000
BN = 2000
GW = 144
SW = 160
SQRT3 = 1.7320508075688772


def _whole(shape):
    nd = len(shape)
    return pl.BlockSpec(shape, lambda i, _nd=nd: (0,) * _nd)


def _rows(bs, cols):
    return pl.BlockSpec((bs, cols), lambda i: (i, 0))


def _phase0_body(ps_ref, pr_ref, feat_ref, s0_ref):
    ps = ps_ref[:, 0:3]
    pr = pr_ref[:, 0:3]
    r = ps - pr
    d = jnp.sqrt(jnp.sum(r * r, axis=1, keepdims=True))
    u = r / (d + 1e-9)
    a1 = SQRT3 * u
    na = jnp.sum(a1 * a1, axis=1, keepdims=True)
    x = jnp.maximum(d, 1e-6)
    coef = jnp.sqrt(2.0 / RC) / x
    rb = jnp.concatenate(
        [coef * jnp.sin((k * jnp.pi / RC) * x) for k in range(1, NB + 1)],
        axis=1)
    feat_ref[...] = jnp.concatenate([a1, na, rb], axis=1)
    ones = jnp.ones_like(na)
    s0_ref[...] = jnp.concatenate([ones, a1], axis=1)


def _phase0(ps, pr):
    return pl.pallas_call(
        _phase0_body,
        grid=(N_EDGES // BE,),
        in_specs=[_rows(BE, 4), _rows(BE, 4)],
        out_specs=[_rows(BE, 8), _rows(BE, 4)],
        out_shape=[jax.ShapeDtypeStruct((N_EDGES, 8), jnp.float32),
                   jax.ShapeDtypeStruct((N_EDGES, 4), jnp.float32)],
    )(ps, pr)


def _prep_body(ns_ref, nv_ref, wes_ref, wev_ref, p_ref):
    mm = jnp.dot(ns_ref[...], wes_ref[...], preferred_element_type=jnp.float32, precision=jax.lax.Precision.HIGHEST)
    vt = wev_ref[0, 0] * nv_ref[:, 0:3]
    pad = jnp.zeros((mm.shape[0], GW - F - 3), jnp.float32)
    p_ref[...] = jnp.concatenate([mm, vt, pad], axis=1)


def _prep(ns, nv4, wes, wev):
    return pl.pallas_call(
        _prep_body,
        grid=(N_NODES // BN,),
        in_specs=[_rows(BN, F), _rows(BN, 4), _whole((F, F)), _whole((1, 1))],
        out_specs=_rows(BN, GW),
        out_shape=jax.ShapeDtypeStruct((N_NODES, GW), jnp.float32),
    )(ns, nv4, wes, wev)


def _edge_body(g_ref, feat_ref, r0_ref, r1_ref, r2_ref, r3s_ref, r3v_ref,
               wmsa_ref, wmsb_ref, wmva_ref, wvrows_ref, s_ref):
    g = g_ref[:, 0:F]
    mv = g_ref[:, F:F + 3]
    a1 = feat_ref[:, 0:3]
    na = feat_ref[:, 3:4]
    rb = feat_ref[:, 4:8]
    h = jax.nn.gelu(jnp.dot(rb, r0_ref[...], preferred_element_type=jnp.float32, precision=jax.lax.Precision.HIGHEST))
    h = jax.nn.gelu(jnp.dot(h, r1_ref[...], preferred_element_type=jnp.float32, precision=jax.lax.Precision.HIGHEST))
    h = jax.nn.gelu(jnp.dot(h, r2_ref[...], preferred_element_type=jnp.float32, precision=jax.lax.Precision.HIGHEST))
    w1 = jnp.dot(h, r3s_ref[...], preferred_element_type=jnp.float32, precision=jax.lax.Precision.HIGHEST)
    w2 = jnp.dot(h, r3v_ref[...], preferred_element_type=jnp.float32, precision=jax.lax.Precision.HIGHEST)
    mva1 = jnp.sum(mv * a1, axis=1, keepdims=True)
    t3 = jnp.concatenate([w1[:, 128:129], w1[:, 129:130] * mva1,
                          w1[:, 130:131] * na], axis=1)
    es = (jnp.dot(w1[:, 0:128] * g, wmsa_ref[...],
                  preferred_element_type=jnp.float32, precision=jax.lax.Precision.HIGHEST)
          + jnp.dot(t3, wmsb_ref[...], preferred_element_type=jnp.float32, precision=jax.lax.Precision.HIGHEST))
    q = (jnp.dot(w2[:, 0:128] * g, wmva_ref[...],
                 preferred_element_type=jnp.float32, precision=jax.lax.Precision.HIGHEST)
         + w2[:, 128:129] * wvrows_ref[0:1, :]
         + w2[:, 130:131] * wvrows_ref[2:3, :])
    pc = w2[:, 129:130] * wvrows_ref[1:2, :]
    evs = [a1[:, i:i + 1] * q + mv[:, i:i + 1] * pc for i in range(3)]
    pad = jnp.zeros((g.shape[0], SW - 85 - 63), jnp.float32)
    s_ref[...] = jnp.concatenate([es] + evs + [pad], axis=1)


def _edge(gath, feat, wp):
    return pl.pallas_call(
        _edge_body,
        grid=(N_EDGES // BE,),
        in_specs=[_rows(BE, GW), _rows(BE, 8),
                  _whole((NB, 128)), _whole((128, 128)), _whole((128, 128)),
                  _whole((128, 131)), _whole((128, 131)),
                  _whole((128, 85)), _whole((3, 85)),
                  _whole((128, 21)), _whole((3, 21))],
        out_specs=_rows(BE, SW),
        out_shape=jax.ShapeDtypeStruct((N_EDGES, SW), jnp.float32),
    )(gath, feat, wp['R0'], wp['R1'], wp['R2'], wp['R3s'], wp['R3v'],
      wp['WmsA'], wp['WmsB'], wp['WmvA'], wp['Wvrows'])


def _upd_body(a_ref, c_ref, ns_ref, nv_ref, wns_ref, wnv_ref, wbs_ref,
              wbv_ref, ns_out, nv_out):
    inv_sqE = 1.0 / jnp.sqrt(float(N_EDGES))
    scale = inv_sqE / jnp.maximum(c_ref[:, 0:1], 1.0)
    a = a_ref[...]
    lin_s = a[:, 0:85] * scale + jnp.dot(ns_ref[...], wns_ref[...],
                                         preferred_element_type=jnp.float32, precision=jax.lax.Precision.HIGHEST)
    gates = jax.nn.sigmoid(lin_s[:, 0:M1])
    scal = jax.nn.gelu(lin_s[:, M1:85])
    ns_out[...] = jnp.dot(scal, wbs_ref[...], preferred_element_type=jnp.float32, precision=jax.lax.Precision.HIGHEST)
    gw = gates * wbv_ref[0:1, :]
    cols = []
    for i in range(3):
        lin_v = a[:, 85 + M1 * i:85 + M1 * (i + 1)] * scale \
            + nv_ref[:, i:i + 1] * wnv_ref[0:1, :]
        cols.append(jnp.sum(gw * lin_v, axis=1, keepdims=True))
    cols.append(jnp.zeros_like(cols[0]))
    nv_out[...] = jnp.concatenate(cols, axis=1)


def _update(agg, cnt4, ns, nv4, wp):
    return pl.pallas_call(
        _upd_body,
        grid=(N_NODES // BN,),
        in_specs=[_rows(BN, SW), _rows(BN, 4), _rows(BN, F), _rows(BN, 4),
                  _whole((F, 85)), _whole((1, M1)), _whole((M0, F)),
                  _whole((1, M1))],
        out_specs=[_rows(BN, F), _rows(BN, 4)],
        out_shape=[jax.ShapeDtypeStruct((N_NODES, F), jnp.float32),
                   jax.ShapeDtypeStruct((N_NODES, 4), jnp.float32)],
    )(agg, cnt4, ns, nv4, wp['Wn_s'], wp['Wnv_row'], wp['Wb_s'], wp['Wbv_row'])


def _readout_body(ns_ref, nv_ref, c_ref, wpa_ref, wpb_ref,
                  w0_ref, b0_ref, w1_ref, b1_ref, w2_ref, b2_ref,
                  w3_ref, b3_ref, out_ref, acc1, acc2):
    i = pl.program_id(0)
    nblk = pl.num_programs(0)

    @pl.when(i == 0)
    def _init():
        acc1[...] = jnp.zeros_like(acc1)
        acc2[0] = 0.0

    attr_s = c_ref[:, 0:1] * (1.0 / N_EDGES)
    attr_v = c_ref[:, 1:4] * (1.0 / N_EDGES)
    acc1[...] += jax.lax.dot_general(
        attr_s, ns_ref[...], (((0,), (0,)), ((), ())),
        preferred_element_type=jnp.float32, precision=jax.lax.Precision.HIGHEST)
    u = jnp.sum(nv_ref[:, 0:3] * attr_v)
    acc2[0] += u

    @pl.when(i == nblk - 1)
    def _fin():
        pre = (jnp.dot(acc1[...] * (1.0 / N_NODES), wpa_ref[...],
                       preferred_element_type=jnp.float32, precision=jax.lax.Precision.HIGHEST)
               + (acc2[0] * (1.0 / N_NODES)) * wpb_ref[...])
        x = jax.nn.gelu(jnp.dot(pre, w0_ref[...],
                                preferred_element_type=jnp.float32, precision=jax.lax.Precision.HIGHEST) + b0_ref[...])
        x = jax.nn.gelu(jnp.dot(x, w1_ref[...],
                                preferred_element_type=jnp.float32, precision=jax.lax.Precision.HIGHEST) + b1_ref[...])
        x = jax.nn.gelu(jnp.dot(x, w2_ref[...],
                                preferred_element_type=jnp.float32, precision=jax.lax.Precision.HIGHEST) + b2_ref[...])
        x = jnp.dot(x, w3_ref[...], preferred_element_type=jnp.float32, precision=jax.lax.Precision.HIGHEST) \
            + b3_ref[...]
        out_ref[...] = x

    _ = (wpa_ref, wpb_ref, w0_ref, b0_ref, w1_ref, b1_ref, w2_ref, b2_ref,
         w3_ref, b3_ref)


def _readout(ns, nv4, cnt4, params):
    mlp = params['mlp']
    wpa = params['W_pre'][:F]
    wpb = params['W_pre'][F:F + 1]
    ops = [ns, nv4, cnt4, wpa, wpb,
           mlp[0]['W'], mlp[0]['b'][None, :], mlp[1]['W'], mlp[1]['b'][None, :],
           mlp[2]['W'], mlp[2]['b'][None, :], mlp[3]['W'], mlp[3]['b'][None, :]]
    in_specs = [_rows(BN, F), _rows(BN, 4), _rows(BN, 4),
                _whole((F, F)), _whole((1, F))]
    for k in range(4):
        in_specs.append(_whole(tuple(mlp[k]['W'].shape)))
        in_specs.append(_whole((1,) + tuple(mlp[k]['b'].shape)))
    out = pl.pallas_call(
        _readout_body,
        grid=(N_NODES // BN,),
        in_specs=in_specs,
        out_specs=pl.BlockSpec((1, 1), lambda i: (0, 0)),
        out_shape=jax.ShapeDtypeStruct((1, 1), jnp.float32),
        scratch_shapes=[pltpu.VMEM((1, F), jnp.float32),
                        pltpu.SMEM((1,), jnp.float32)],
    )(*ops)
    return out[0]


def kernel(nodes_s, nodes_v, senders, receivers, params):
    senders = senders.astype(jnp.int32)
    receivers = receivers.astype(jnp.int32)
    nv4 = jnp.pad(nodes_v[:, 0, :], ((0, 0), (0, 1)))

    ps = nv4[senders]
    pr = nv4[receivers]
    feat, s0 = _phase0(ps, pr)
    cnt4 = jax.ops.segment_sum(s0, receivers, num_segments=N_NODES)

    ns = nodes_s
    nv = nv4
    for t in range(STEPS):
        p = params['s%d' % t]
        wp = {
            'R0': p['R0'], 'R1': p['R1'], 'R2': p['R2'],
            'R3s': p['R3'][:, :131], 'R3v': p['R3'][:, 131:],
            'WmsA': p['Wm_s'][:128], 'WmsB': p['Wm_s'][128:131],
            'WmvA': p['Wm_v'][:128], 'Wvrows': p['Wm_v'][128:131],
            'Wn_s': p['Wn_s'], 'Wnv_row': p['Wn_v'][0:1, :],
            'Wb_s': p['Wb_s'], 'Wbv_row': p['Wb_v'][:, 0][None, :],
        }
        payload = _prep(ns, nv, p['W_es'], p['W_ev'])
        gath = payload[senders]
        s = _edge(gath, feat, wp)
        agg = jax.ops.segment_sum(s, receivers, num_segments=N_NODES)
        ns, nv = _update(agg, cnt4, ns, nv, wp)

    return _readout(ns, nv, cnt4, params)


# speedup vs baseline: 10.0055x; 1.2587x over previous
import functools
import jax
import jax.numpy as jnp
from jax import lax
from jax.experimental import pallas as pl
from jax.experimental.pallas import tpu as pltpu
from jax.experimental.pallas import tpu_sc as plsc

N_NODES = 10000
N_EDGES = 320000
F = 128
NB = 4
RC = 1.0
M0 = 64
M1 = 21
STEPS = 3

BE = 2000
BN = 2000
SW = 160
SQRT3 = 1.7320508075688772

CHUNK = 80
NW = 32
E2 = N_EDGES // CHUNK
PER_W = E2 // NW
PER_S = E2 // 16
NP = 10240
NHALF = NP // 2
ACC_R = NHALF + 128
ACC_S = ACC_R // 16
OUT_S = NHALF // 16
GARB = NHALF + 64
TABW = 40960

_SC_PARAMS = pltpu.CompilerParams(needs_layout_passes=False)


def _sc_mesh():
    return plsc.VectorSubcoreMesh(core_axis_name="c", subcore_axis_name="s")


def _sc_gather(payload, tab, idx3):
    @functools.partial(
        pl.kernel, mesh=_sc_mesh(),
        out_type=[jax.ShapeDtypeStruct((N_EDGES, F), jnp.float32),
                  jax.ShapeDtypeStruct((N_EDGES, 8), jnp.float32)],
        scratch_types=[pltpu.VMEM((PER_W, CHUNK), jnp.int32),
                       pltpu.VMEM((CHUNK, F), jnp.float32),
                       pltpu.VMEM((TABW,), jnp.float32),
                       pltpu.VMEM((CHUNK, 8), jnp.float32),
                       pltpu.SemaphoreType.DMA],
        compiler_params=_SC_PARAMS)
    def k(p_hbm, tab_hbm, idx_hbm, out_hbm, mv_hbm,
          idx_v, rows_v, tab_v, mv_v, sem):
        wid = lax.axis_index("s") * 2 + lax.axis_index("c")
        base = wid * PER_W
        pltpu.sync_copy(tab_hbm, tab_v)
        pltpu.sync_copy(idx_hbm.at[wid], idx_v)
        lane = lax.iota(jnp.int32, 16)

        @pl.loop(0, PER_W)
        def _chunk(j):
            pltpu.async_copy(p_hbm.at[idx_v.at[j]], rows_v, sem).wait()
            for t in range(CHUNK // 16):
                i16 = idx_v[j, pl.ds(t * 16, 16)]
                for c in range(3):
                    vals = plsc.load_gather(tab_v, [i16 * 4 + c])
                    plsc.store_scatter(
                        mv_v, [lane + t * 16, jnp.full((16,), c, jnp.int32)],
                        vals)
            pltpu.sync_copy(rows_v, out_hbm.at[pl.ds((base + j) * CHUNK, CHUNK)])
            pltpu.sync_copy(mv_v, mv_hbm.at[pl.ds((base + j) * CHUNK, CHUNK)])

    return k(payload, tab, idx3)


def _sc_edge_vec(tab, sidx3, ridx3):
    @functools.partial(
        pl.kernel, mesh=_sc_mesh(),
        out_type=jax.ShapeDtypeStruct((N_EDGES, 8), jnp.float32),
        scratch_types=[pltpu.VMEM((PER_W, CHUNK), jnp.int32),
                       pltpu.VMEM((PER_W, CHUNK), jnp.int32),
                       pltpu.VMEM((TABW,), jnp.float32),
                       pltpu.VMEM((CHUNK, 8), jnp.float32),
                       pltpu.SemaphoreType.DMA],
        compiler_params=_SC_PARAMS)
    def k(tab_hbm, s_hbm, r_hbm, out_hbm, sidx_v, ridx_v, tab_v, r8_v, sem):
        wid = lax.axis_index("s") * 2 + lax.axis_index("c")
        base = wid * PER_W
        pltpu.sync_copy(tab_hbm, tab_v)
        pltpu.sync_copy(s_hbm.at[wid], sidx_v)
        pltpu.sync_copy(r_hbm.at[wid], ridx_v)
        lane = lax.iota(jnp.int32, 16)

        @pl.loop(0, PER_W)
        def _chunk(j):
            for t in range(CHUNK // 16):
                s16 = sidx_v[j, pl.ds(t * 16, 16)]
                r16 = ridx_v[j, pl.ds(t * 16, 16)]
                for c in range(3):
                    vs = plsc.load_gather(tab_v, [s16 * 4 + c])
                    vr = plsc.load_gather(tab_v, [r16 * 4 + c])
                    plsc.store_scatter(
                        r8_v, [lane + t * 16, jnp.full((16,), c, jnp.int32)],
                        vs - vr)
            pltpu.sync_copy(r8_v, out_hbm.at[pl.ds((base + j) * CHUNK, CHUNK)])

    return k(tab, sidx3, ridx3)


def _sc_scatter2(s1, s2, idx16, zeros):
    @functools.partial(
        pl.kernel, mesh=_sc_mesh(),
        out_type=[jax.ShapeDtypeStruct((NP, F), jnp.float32),
                  jax.ShapeDtypeStruct((NP, F), jnp.float32)],
        scratch_types=[pltpu.VMEM((PER_S, CHUNK), jnp.int32),
                       pltpu.VMEM((PER_S, CHUNK), jnp.int32),
                       pltpu.VMEM((CHUNK, F), jnp.float32),
                       pltpu.VMEM_SHARED((ACC_R, F), jnp.float32),
                       pltpu.SemaphoreType.DMA],
        compiler_params=_SC_PARAMS)
    def k(s1_hbm, s2_hbm, idx_hbm, z_hbm, out1_hbm, out2_hbm,
          idx_v, idx2_v, rows_v, acc, sem):
        cid = lax.axis_index("c")
        sid = lax.axis_index("s")
        nbase = cid * NHALF
        pltpu.sync_copy(idx_hbm.at[sid], idx_v)

        @pl.loop(0, PER_S)
        def _remap(r):
            for t in range(CHUNK // 16):
                i16 = idx_v[r, pl.ds(t * 16, 16)]
                rel = i16 - nbase
                ok = (rel >= 0) & (rel < NHALF)
                idx2_v[r, pl.ds(t * 16, 16)] = jnp.where(
                    ok, rel, jnp.full((16,), GARB, jnp.int32))

        for (src, dst) in ((s1_hbm, out1_hbm), (s2_hbm, out2_hbm)):
            pltpu.sync_copy(z_hbm.at[pl.ds(sid * ACC_S, ACC_S)],
                            acc.at[pl.ds(sid * ACC_S, ACC_S)])
            plsc.subcore_barrier()

            @pl.loop(0, PER_S)
            def _chunk(r):
                pltpu.sync_copy(src.at[pl.ds((sid * PER_S + r) * CHUNK, CHUNK)],
                                rows_v)
                pltpu.sync_copy(rows_v, acc.at[idx2_v.at[r]], add=True)

            plsc.subcore_barrier()
            pltpu.sync_copy(
                acc.at[pl.ds(sid * OUT_S, OUT_S)],
                dst.at[pl.ds(nbase + sid * OUT_S, OUT_S)])
            plsc.subcore_barrier()

    return k(s1, s2, idx16, zeros)


def _whole(shape):
    nd = len(shape)
    return pl.BlockSpec(shape, lambda i, _nd=nd: (0,) * _nd)


def _rows(bs, cols):
    return pl.BlockSpec((bs, cols), lambda i: (i, 0))


_HI = jax.lax.Precision.HIGHEST


def _dot(a, b):
    return jnp.dot(a, b, preferred_element_type=jnp.float32, precision=_HI)


def _phase0_body(r8_ref, feat_ref):
    r = r8_ref[:, 0:3]
    d = jnp.sqrt(jnp.sum(r * r, axis=1, keepdims=True))
    u = r / (d + 1e-9)
    a1 = SQRT3 * u
    na = jnp.sum(a1 * a1, axis=1, keepdims=True)
    x = jnp.maximum(d, 1e-6)
    coef = jnp.sqrt(2.0 / RC) / x
    rb = jnp.concatenate(
        [coef * jnp.sin((k * jnp.pi / RC) * x) for k in range(1, NB + 1)],
        axis=1)
    feat_ref[...] = jnp.concatenate([a1, na, rb], axis=1)


def _phase0(r8):
    return pl.pallas_call(
        _phase0_body,
        grid=(N_EDGES // BE,),
        in_specs=[_rows(BE, 8)],
        out_specs=_rows(BE, 8),
        out_shape=jax.ShapeDtypeStruct((N_EDGES, 8), jnp.float32),
    )(r8)


def _prep_body(ns_ref, wes_ref, p_ref):
    p_ref[...] = _dot(ns_ref[...], wes_ref[...])


def _prep(ns, wes):
    return pl.pallas_call(
        _prep_body,
        grid=(N_NODES // BN,),
        in_specs=[_rows(BN, F), _whole((F, F))],
        out_specs=_rows(BN, F),
        out_shape=jax.ShapeDtypeStruct((N_NODES, F), jnp.float32),
    )(ns, wes)


def _edge_body(g_ref, mv_ref, feat_ref, r0_ref, r1_ref, r2_ref, r3s_ref,
               r3v_ref, wmsa_ref, wmsb_ref, wmva_ref, wvrows_ref, wev_ref,
               s1_ref):
    g = g_ref[...]
    mv = wev_ref[0, 0] * mv_ref[:, 0:3]
    a1 = feat_ref[:, 0:3]
    na = feat_ref[:, 3:4]
    rb = feat_ref[:, 4:8]
    h = jax.nn.gelu(_dot(rb, r0_ref[...]))
    h = jax.nn.gelu(_dot(h, r1_ref[...]))
    h = jax.nn.gelu(_dot(h, r2_ref[...]))
    w1 = _dot(h, r3s_ref[...])
    w2 = _dot(h, r3v_ref[...])
    mva1 = jnp.sum(mv * a1, axis=1, keepdims=True)
    t3 = jnp.concatenate([w1[:, 128:129], w1[:, 129:130] * mva1,
                          w1[:, 130:131] * na], axis=1)
    es = _dot(w1[:, 0:128] * g, wmsa_ref[...]) + _dot(t3, wmsb_ref[...])
    q = (_dot(w2[:, 0:128] * g, wmva_ref[...])
         + w2[:, 128:129] * wvrows_ref[0:1, :]
         + w2[:, 130:131] * wvrows_ref[2:3, :])
    pc = w2[:, 129:130] * wvrows_ref[1:2, :]
    evs = [a1[:, i:i + 1] * q + mv[:, i:i + 1] * pc for i in range(3)]
    ones = jnp.ones((g.shape[0], 1), jnp.float32)
    pad = jnp.zeros((g.shape[0], SW - 152), jnp.float32)
    s1_ref[...] = jnp.concatenate(
        [es, evs[0], evs[1], evs[2], ones, a1, pad], axis=1)


def _edge(gath, mv8, feat, wp):
    return pl.pallas_call(
        _edge_body,
        grid=(N_EDGES // BE,),
        in_specs=[_rows(BE, F), _rows(BE, 8), _rows(BE, 8),
                  _whole((NB, 128)), _whole((128, 128)), _whole((128, 128)),
                  _whole((128, 131)), _whole((128, 131)),
                  _whole((128, 85)), _whole((3, 85)),
                  _whole((128, 21)), _whole((3, 21)), _whole((1, 1))],
        out_specs=_rows(BE, SW),
        out_shape=jax.ShapeDtypeStruct((N_EDGES, SW), jnp.float32),
    )(gath, mv8, feat, wp['R0'], wp['R1'], wp['R2'], wp['R3s'], wp['R3v'],
      wp['WmsA'], wp['WmsB'], wp['WmvA'], wp['Wvrows'], wp['W_ev'])


def _upd_body(a1_ref, c_ref, ns_ref, nv_ref, wns_ref, wnv_ref,
              wbs_ref, wbv_ref, ns_out, nv_out):
    inv_sqE = 1.0 / jnp.sqrt(float(N_EDGES))
    scale = inv_sqE / jnp.maximum(c_ref[:, 148:149], 1.0)
    a1 = a1_ref[...]
    lin_s = a1[:, 0:85] * scale + _dot(ns_ref[...], wns_ref[...])
    gates = jax.nn.sigmoid(lin_s[:, 0:M1])
    scal = jax.nn.gelu(lin_s[:, M1:85])
    ns_out[...] = _dot(scal, wbs_ref[...])
    gw = gates * wbv_ref[0:1, :]
    evs = [a1[:, 85:106], a1[:, 106:127], a1[:, 127:148]]
    cols = []
    for i in range(3):
        lin_v = evs[i] * scale + nv_ref[:, i:i + 1] * wnv_ref[0:1, :]
        cols.append(jnp.sum(gw * lin_v, axis=1, keepdims=True))
    cols.append(jnp.zeros_like(cols[0]))
    nv_out[...] = jnp.concatenate(cols, axis=1)


def _update(agg1, cnt_src, ns, nv4, wp):
    return pl.pallas_call(
        _upd_body,
        grid=(N_NODES // BN,),
        in_specs=[_rows(BN, SW), _rows(BN, SW),
                  _rows(BN, F), _rows(BN, 4),
                  _whole((F, 85)), _whole((1, M1)), _whole((M0, F)),
                  _whole((1, M1))],
        out_specs=[_rows(BN, F), _rows(BN, 4)],
        out_shape=[jax.ShapeDtypeStruct((N_NODES, F), jnp.float32),
                   jax.ShapeDtypeStruct((N_NODES, 4), jnp.float32)],
    )(agg1, cnt_src, ns, nv4,
      wp['Wn_s'], wp['Wnv_row'], wp['Wb_s'], wp['Wbv_row'])


def _readout_body(ns_ref, nv_ref, c1_ref, wpa_ref, wpb_ref,
                  w0_ref, b0_ref, w1_ref, b1_ref, w2_ref, b2_ref,
                  w3_ref, b3_ref, out_ref, acc1, acc2):
    i = pl.program_id(0)
    nblk = pl.num_programs(0)

    @pl.when(i == 0)
    def _init():
        acc1[...] = jnp.zeros_like(acc1)
        acc2[0] = 0.0

    attr_s = c1_ref[:, 148:149] * (1.0 / N_EDGES)
    attr_v = c1_ref[:, 149:152] * (1.0 / N_EDGES)
    acc1[...] += jax.lax.dot_general(
        attr_s, ns_ref[...], (((0,), (0,)), ((), ())),
        preferred_element_type=jnp.float32, precision=_HI)
    u = jnp.sum(nv_ref[:, 0:3] * attr_v)
    acc2[0] += u

    @pl.when(i == nblk - 1)
    def _fin():
        pre = (_dot(acc1[...] * (1.0 / N_NODES), wpa_ref[...])
               + (acc2[0] * (1.0 / N_NODES)) * wpb_ref[...])
        x = jax.nn.gelu(_dot(pre, w0_ref[...]) + b0_ref[...])
        x = jax.nn.gelu(_dot(x, w1_ref[...]) + b1_ref[...])
        x = jax.nn.gelu(_dot(x, w2_ref[...]) + b2_ref[...])
        x = _dot(x, w3_ref[...]) + b3_ref[...]
        out_ref[...] = x


def _readout(ns, nv4, cnt1, params):
    mlp = params['mlp']
    wpa = params['W_pre'][:F]
    wpb = params['W_pre'][F:F + 1]
    ops = [ns, nv4, cnt1, wpa, wpb,
           mlp[0]['W'], mlp[0]['b'][None, :], mlp[1]['W'], mlp[1]['b'][None, :],
           mlp[2]['W'], mlp[2]['b'][None, :], mlp[3]['W'], mlp[3]['b'][None, :]]
    in_specs = [_rows(BN, F), _rows(BN, 4), _rows(BN, SW),
                _whole((F, F)), _whole((1, F))]
    for k in range(4):
        in_specs.append(_whole(tuple(mlp[k]['W'].shape)))
        in_specs.append(_whole((1,) + tuple(mlp[k]['b'].shape)))
    out = pl.pallas_call(
        _readout_body,
        grid=(N_NODES // BN,),
        in_specs=in_specs,
        out_specs=pl.BlockSpec((1, 1), lambda i: (0, 0)),
        out_shape=jax.ShapeDtypeStruct((1, 1), jnp.float32),
        scratch_shapes=[pltpu.VMEM((1, F), jnp.float32),
                        pltpu.SMEM((1,), jnp.float32)],
    )(*ops)
    return out[0]


def kernel(nodes_s, nodes_v, senders, receivers, params):
    senders = senders.astype(jnp.int32)
    receivers = receivers.astype(jnp.int32)
    sen3 = senders.reshape(NW, PER_W, CHUNK)
    rec3 = receivers.reshape(NW, PER_W, CHUNK)
    nv4 = jnp.pad(nodes_v[:, 0, :], ((0, 0), (0, 1)))

    r8 = _sc_edge_vec(jnp.pad(nv4.reshape(-1), (0, TABW - 4 * N_NODES)),
                      sen3, rec3)
    feat = _phase0(r8)

    ns = nodes_s
    nv = nv4
    cnt1 = None
    cnt2 = None
    for t in range(STEPS):
        p = params['s%d' % t]
        wp = {
            'R0': p['R0'], 'R1': p['R1'], 'R2': p['R2'],
            'R3s': p['R3'][:, :131], 'R3v': p['R3'][:, 131:],
            'WmsA': p['Wm_s'][:128], 'WmsB': p['Wm_s'][128:131],
            'WmvA': p['Wm_v'][:128], 'Wvrows': p['Wm_v'][128:131],
            'Wn_s': p['Wn_s'], 'Wnv_row': p['Wn_v'][0:1, :],
            'Wb_s': p['Wb_s'], 'Wbv_row': p['Wb_v'][:, 0][None, :],
            'W_ev': p['W_ev'],
        }
        payload = _prep(ns, p['W_es'])
        tab = jnp.pad(nv.reshape(-1), (0, TABW - 4 * N_NODES))
        gath, mv8 = _sc_gather(payload, tab, sen3)
        s1 = _edge(gath, mv8, feat, wp)
        agg1 = jax.ops.segment_sum(s1, receivers, num_segments=N_NODES)
        if t == 0:
            cnt1 = agg1
        ns, nv = _update(agg1, cnt1, ns, nv, wp)

    return _readout(ns, nv, cnt1, params)


# speedup vs baseline: 10.7890x; 1.0783x over previous
import functools
import jax
import jax.numpy as jnp
from jax import lax
from jax.experimental import pallas as pl
from jax.experimental.pallas import tpu as pltpu
from jax.experimental.pallas import tpu_sc as plsc

N_NODES = 10000
N_EDGES = 320000
F = 128
NB = 4
RC = 1.0
M0 = 64
M1 = 21
STEPS = 3

BE = 4000
BN = 2000
SW = 152
SQRT3 = 1.7320508075688772

CHUNK = 80
NW = 32
E2 = N_EDGES // CHUNK
PER_W = E2 // NW
PER_S = E2 // 16
NP = 10240
NHALF = NP // 2
ACC_R = NHALF + 128
ACC_S = ACC_R // 16
OUT_S = NHALF // 16
GARB = NHALF + 64
TABW = 40960

_SC_PARAMS = pltpu.CompilerParams(needs_layout_passes=False)


def _sc_mesh():
    return plsc.VectorSubcoreMesh(core_axis_name="c", subcore_axis_name="s")


def _sc_gather(payload, tab, idx3):
    @functools.partial(
        pl.kernel, mesh=_sc_mesh(),
        out_type=[jax.ShapeDtypeStruct((N_EDGES, F), jnp.float32),
                  jax.ShapeDtypeStruct((N_EDGES, 8), jnp.float32)],
        scratch_types=[pltpu.VMEM((PER_W, CHUNK), jnp.int32),
                       pltpu.VMEM((CHUNK, F), jnp.float32),
                       pltpu.VMEM((TABW,), jnp.float32),
                       pltpu.VMEM((CHUNK, 8), jnp.float32),
                       pltpu.SemaphoreType.DMA],
        compiler_params=_SC_PARAMS)
    def k(p_hbm, tab_hbm, idx_hbm, out_hbm, mv_hbm,
          idx_v, rows_v, tab_v, mv_v, sem):
        wid = lax.axis_index("s") * 2 + lax.axis_index("c")
        base = wid * PER_W
        pltpu.sync_copy(tab_hbm, tab_v)
        pltpu.sync_copy(idx_hbm.at[wid], idx_v)
        lane = lax.iota(jnp.int32, 16)

        @pl.loop(0, PER_W)
        def _chunk(j):
            pltpu.async_copy(p_hbm.at[idx_v.at[j]], rows_v, sem).wait()
            for t in range(CHUNK // 16):
                i16 = idx_v[j, pl.ds(t * 16, 16)]
                for c in range(3):
                    vals = plsc.load_gather(tab_v, [i16 * 4 + c])
                    plsc.store_scatter(
                        mv_v, [lane + t * 16, jnp.full((16,), c, jnp.int32)],
                        vals)
            pltpu.sync_copy(rows_v, out_hbm.at[pl.ds((base + j) * CHUNK, CHUNK)])
            pltpu.sync_copy(mv_v, mv_hbm.at[pl.ds((base + j) * CHUNK, CHUNK)])

    return k(payload, tab, idx3)


def _sc_edge_vec(tab, sidx3, ridx3):
    @functools.partial(
        pl.kernel, mesh=_sc_mesh(),
        out_type=jax.ShapeDtypeStruct((N_EDGES, 8), jnp.float32),
        scratch_types=[pltpu.VMEM((PER_W, CHUNK), jnp.int32),
                       pltpu.VMEM((PER_W, CHUNK), jnp.int32),
                       pltpu.VMEM((TABW,), jnp.float32),
                       pltpu.VMEM((CHUNK, 8), jnp.float32),
                       pltpu.SemaphoreType.DMA],
        compiler_params=_SC_PARAMS)
    def k(tab_hbm, s_hbm, r_hbm, out_hbm, sidx_v, ridx_v, tab_v, r8_v, sem):
        wid = lax.axis_index("s") * 2 + lax.axis_index("c")
        base = wid * PER_W
        pltpu.sync_copy(tab_hbm, tab_v)
        pltpu.sync_copy(s_hbm.at[wid], sidx_v)
        pltpu.sync_copy(r_hbm.at[wid], ridx_v)
        lane = lax.iota(jnp.int32, 16)

        @pl.loop(0, PER_W)
        def _chunk(j):
            for t in range(CHUNK // 16):
                s16 = sidx_v[j, pl.ds(t * 16, 16)]
                r16 = ridx_v[j, pl.ds(t * 16, 16)]
                for c in range(3):
                    vs = plsc.load_gather(tab_v, [s16 * 4 + c])
                    vr = plsc.load_gather(tab_v, [r16 * 4 + c])
                    plsc.store_scatter(
                        r8_v, [lane + t * 16, jnp.full((16,), c, jnp.int32)],
                        vs - vr)
            pltpu.sync_copy(r8_v, out_hbm.at[pl.ds((base + j) * CHUNK, CHUNK)])

    return k(tab, sidx3, ridx3)


def _sc_scatter2(s1, s2, idx16, zeros):
    @functools.partial(
        pl.kernel, mesh=_sc_mesh(),
        out_type=[jax.ShapeDtypeStruct((NP, F), jnp.float32),
                  jax.ShapeDtypeStruct((NP, F), jnp.float32)],
        scratch_types=[pltpu.VMEM((PER_S, CHUNK), jnp.int32),
                       pltpu.VMEM((PER_S, CHUNK), jnp.int32),
                       pltpu.VMEM((CHUNK, F), jnp.float32),
                       pltpu.VMEM_SHARED((ACC_R, F), jnp.float32),
                       pltpu.SemaphoreType.DMA],
        compiler_params=_SC_PARAMS)
    def k(s1_hbm, s2_hbm, idx_hbm, z_hbm, out1_hbm, out2_hbm,
          idx_v, idx2_v, rows_v, acc, sem):
        cid = lax.axis_index("c")
        sid = lax.axis_index("s")
        nbase = cid * NHALF
        pltpu.sync_copy(idx_hbm.at[sid], idx_v)

        @pl.loop(0, PER_S)
        def _remap(r):
            for t in range(CHUNK // 16):
                i16 = idx_v[r, pl.ds(t * 16, 16)]
                rel = i16 - nbase
                ok = (rel >= 0) & (rel < NHALF)
                idx2_v[r, pl.ds(t * 16, 16)] = jnp.where(
                    ok, rel, jnp.full((16,), GARB, jnp.int32))

        for (src, dst) in ((s1_hbm, out1_hbm), (s2_hbm, out2_hbm)):
            pltpu.sync_copy(z_hbm.at[pl.ds(sid * ACC_S, ACC_S)],
                            acc.at[pl.ds(sid * ACC_S, ACC_S)])
            plsc.subcore_barrier()

            @pl.loop(0, PER_S)
            def _chunk(r):
                pltpu.sync_copy(src.at[pl.ds((sid * PER_S + r) * CHUNK, CHUNK)],
                                rows_v)
                pltpu.sync_copy(rows_v, acc.at[idx2_v.at[r]], add=True)

            plsc.subcore_barrier()
            pltpu.sync_copy(
                acc.at[pl.ds(sid * OUT_S, OUT_S)],
                dst.at[pl.ds(nbase + sid * OUT_S, OUT_S)])
            plsc.subcore_barrier()

    return k(s1, s2, idx16, zeros)


def _whole(shape):
    nd = len(shape)
    return pl.BlockSpec(shape, lambda i, _nd=nd: (0,) * _nd)


def _rows(bs, cols):
    return pl.BlockSpec((bs, cols), lambda i: (i, 0))


_HI = jax.lax.Precision.HIGHEST


def _dot(a, b):
    return jnp.dot(a, b, preferred_element_type=jnp.float32, precision=_HI)


def _doth(a, b):
    return jnp.dot(a, b, preferred_element_type=jnp.float32,
                   precision=_HI)


def _phase0_body(r8_ref, feat_ref):
    r = r8_ref[:, 0:3]
    d = jnp.sqrt(jnp.sum(r * r, axis=1, keepdims=True))
    u = r / (d + 1e-9)
    a1 = SQRT3 * u
    na = jnp.sum(a1 * a1, axis=1, keepdims=True)
    x = jnp.maximum(d, 1e-6)
    coef = jnp.sqrt(2.0 / RC) / x
    rb = jnp.concatenate(
        [coef * jnp.sin((k * jnp.pi / RC) * x) for k in range(1, NB + 1)],
        axis=1)
    feat_ref[...] = jnp.concatenate([a1, na, rb], axis=1)


def _phase0(r8):
    return pl.pallas_call(
        _phase0_body,
        grid=(N_EDGES // BE,),
        in_specs=[_rows(BE, 8)],
        out_specs=_rows(BE, 8),
        out_shape=jax.ShapeDtypeStruct((N_EDGES, 8), jnp.float32),
    )(r8)


def _prep_body(ns_ref, wes_ref, p_ref):
    p_ref[...] = _dot(ns_ref[...], wes_ref[...])


def _prep(ns, wes):
    return pl.pallas_call(
        _prep_body,
        grid=(N_NODES // BN,),
        in_specs=[_rows(BN, F), _whole((F, F))],
        out_specs=_rows(BN, F),
        out_shape=jax.ShapeDtypeStruct((N_NODES, F), jnp.float32),
    )(ns, wes)


def _edge_body(g_ref, mv_ref, feat_ref, r0_ref, r1_ref, r2_ref, w3m_ref,
               w3x_ref, wmsa_ref, wmsb_ref, wmva_ref, wvrows_ref, wev_ref,
               s1_ref):
    g = g_ref[...]
    mv = wev_ref[0, 0] * mv_ref[:, 0:3]
    a1 = feat_ref[:, 0:3]
    na = feat_ref[:, 3:4]
    rb = feat_ref[:, 4:8]
    h = jax.nn.gelu(_doth(rb, r0_ref[...]))
    h = jax.nn.gelu(_doth(h, r1_ref[...]))
    h = jax.nn.gelu(_doth(h, r2_ref[...]))
    wm = _doth(h, w3m_ref[...])
    wx = _doth(h, w3x_ref[...])
    mva1 = jnp.sum(mv * a1, axis=1, keepdims=True)
    t3 = jnp.concatenate([wx[:, 0:1], wx[:, 1:2] * mva1,
                          wx[:, 2:3] * na], axis=1)
    es = _doth(wm[:, 0:128] * g, wmsa_ref[...]) + _doth(t3, wmsb_ref[...])
    q = (_doth(wm[:, 128:256] * g, wmva_ref[...])
         + wx[:, 3:4] * wvrows_ref[0:1, :]
         + wx[:, 5:6] * wvrows_ref[2:3, :])
    pc = wx[:, 4:5] * wvrows_ref[1:2, :]
    evs = [a1[:, i:i + 1] * q + mv[:, i:i + 1] * pc for i in range(3)]
    ones = jnp.ones((g.shape[0], 1), jnp.float32)
    s1_ref[...] = jnp.concatenate(
        [es, evs[0], evs[1], evs[2], ones, a1], axis=1)


def _edge(gath, mv8, feat, wp):
    return pl.pallas_call(
        _edge_body,
        grid=(N_EDGES // BE,),
        in_specs=[_rows(BE, F), _rows(BE, 8), _rows(BE, 8),
                  _whole((NB, 128)), _whole((128, 128)), _whole((128, 128)),
                  _whole((128, 256)), _whole((128, 6)),
                  _whole((128, 85)), _whole((3, 85)),
                  _whole((128, 21)), _whole((3, 21)), _whole((1, 1))],
        out_specs=_rows(BE, SW),
        out_shape=jax.ShapeDtypeStruct((N_EDGES, SW), jnp.float32),
    )(gath, mv8, feat, wp['R0'], wp['R1'], wp['R2'], wp['W3m'], wp['W3x'],
      wp['WmsA'], wp['WmsB'], wp['WmvA'], wp['Wvrows'], wp['W_ev'])


def _upd_body(a1_ref, c_ref, ns_ref, nv_ref, wns_ref, wnv_ref,
              wbs_ref, wbv_ref, ns_out, nv_out):
    inv_sqE = 1.0 / jnp.sqrt(float(N_EDGES))
    scale = inv_sqE / jnp.maximum(c_ref[:, 148:149], 1.0)
    a1 = a1_ref[...]
    lin_s = a1[:, 0:85] * scale + _dot(ns_ref[...], wns_ref[...])
    gates = jax.nn.sigmoid(lin_s[:, 0:M1])
    scal = jax.nn.gelu(lin_s[:, M1:85])
    ns_out[...] = _dot(scal, wbs_ref[...])
    gw = gates * wbv_ref[0:1, :]
    evs = [a1[:, 85:106], a1[:, 106:127], a1[:, 127:148]]
    cols = []
    for i in range(3):
        lin_v = evs[i] * scale + nv_ref[:, i:i + 1] * wnv_ref[0:1, :]
        cols.append(jnp.sum(gw * lin_v, axis=1, keepdims=True))
    cols.append(jnp.zeros_like(cols[0]))
    nv_out[...] = jnp.concatenate(cols, axis=1)


def _update(agg1, cnt_src, ns, nv4, wp):
    return pl.pallas_call(
        _upd_body,
        grid=(N_NODES // BN,),
        in_specs=[_rows(BN, SW), _rows(BN, SW),
                  _rows(BN, F), _rows(BN, 4),
                  _whole((F, 85)), _whole((1, M1)), _whole((M0, F)),
                  _whole((1, M1))],
        out_specs=[_rows(BN, F), _rows(BN, 4)],
        out_shape=[jax.ShapeDtypeStruct((N_NODES, F), jnp.float32),
                   jax.ShapeDtypeStruct((N_NODES, 4), jnp.float32)],
    )(agg1, cnt_src, ns, nv4,
      wp['Wn_s'], wp['Wnv_row'], wp['Wb_s'], wp['Wbv_row'])


def _readout_body(ns_ref, nv_ref, c1_ref, wpa_ref, wpb_ref,
                  w0_ref, b0_ref, w1_ref, b1_ref, w2_ref, b2_ref,
                  w3_ref, b3_ref, out_ref, acc1, acc2):
    i = pl.program_id(0)
    nblk = pl.num_programs(0)

    @pl.when(i == 0)
    def _init():
        acc1[...] = jnp.zeros_like(acc1)
        acc2[0] = 0.0

    attr_s = c1_ref[:, 148:149] * (1.0 / N_EDGES)
    attr_v = c1_ref[:, 149:152] * (1.0 / N_EDGES)
    acc1[...] += jax.lax.dot_general(
        attr_s, ns_ref[...], (((0,), (0,)), ((), ())),
        preferred_element_type=jnp.float32, precision=_HI)
    u = jnp.sum(nv_ref[:, 0:3] * attr_v)
    acc2[0] += u

    @pl.when(i == nblk - 1)
    def _fin():
        pre = (_dot(acc1[...] * (1.0 / N_NODES), wpa_ref[...])
               + (acc2[0] * (1.0 / N_NODES)) * wpb_ref[...])
        x = jax.nn.gelu(_dot(pre, w0_ref[...]) + b0_ref[...])
        x = jax.nn.gelu(_dot(x, w1_ref[...]) + b1_ref[...])
        x = jax.nn.gelu(_dot(x, w2_ref[...]) + b2_ref[...])
        x = _dot(x, w3_ref[...]) + b3_ref[...]
        out_ref[...] = x


def _readout(ns, nv4, cnt1, params):
    mlp = params['mlp']
    wpa = params['W_pre'][:F]
    wpb = params['W_pre'][F:F + 1]
    ops = [ns, nv4, cnt1, wpa, wpb,
           mlp[0]['W'], mlp[0]['b'][None, :], mlp[1]['W'], mlp[1]['b'][None, :],
           mlp[2]['W'], mlp[2]['b'][None, :], mlp[3]['W'], mlp[3]['b'][None, :]]
    in_specs = [_rows(BN, F), _rows(BN, 4), _rows(BN, SW),
                _whole((F, F)), _whole((1, F))]
    for k in range(4):
        in_specs.append(_whole(tuple(mlp[k]['W'].shape)))
        in_specs.append(_whole((1,) + tuple(mlp[k]['b'].shape)))
    out = pl.pallas_call(
        _readout_body,
        grid=(N_NODES // BN,),
        in_specs=in_specs,
        out_specs=pl.BlockSpec((1, 1), lambda i: (0, 0)),
        out_shape=jax.ShapeDtypeStruct((1, 1), jnp.float32),
        scratch_shapes=[pltpu.VMEM((1, F), jnp.float32),
                        pltpu.SMEM((1,), jnp.float32)],
    )(*ops)
    return out[0]


def kernel(nodes_s, nodes_v, senders, receivers, params):
    senders = senders.astype(jnp.int32)
    receivers = receivers.astype(jnp.int32)
    sen3 = senders.reshape(NW, PER_W, CHUNK)
    rec3 = receivers.reshape(NW, PER_W, CHUNK)
    nv4 = jnp.pad(nodes_v[:, 0, :], ((0, 0), (0, 1)))

    r8 = _sc_edge_vec(jnp.pad(nv4.reshape(-1), (0, TABW - 4 * N_NODES)),
                      sen3, rec3)
    feat = _phase0(r8)

    ns = nodes_s
    nv = nv4
    cnt1 = None
    cnt2 = None
    for t in range(STEPS):
        p = params['s%d' % t]
        wp = {
            'R0': p['R0'], 'R1': p['R1'], 'R2': p['R2'],
            'W3m': jnp.concatenate([p['R3'][:, 0:128], p['R3'][:, 131:259]],
                                   axis=1),
            'W3x': jnp.concatenate([p['R3'][:, 128:131], p['R3'][:, 259:262]],
                                   axis=1),
            'WmsA': p['Wm_s'][:128], 'WmsB': p['Wm_s'][128:131],
            'WmvA': p['Wm_v'][:128], 'Wvrows': p['Wm_v'][128:131],
            'Wn_s': p['Wn_s'], 'Wnv_row': p['Wn_v'][0:1, :],
            'Wb_s': p['Wb_s'], 'Wbv_row': p['Wb_v'][:, 0][None, :],
            'W_ev': p['W_ev'],
        }
        payload = _prep(ns, p['W_es'])
        tab = jnp.pad(nv.reshape(-1), (0, TABW - 4 * N_NODES))
        gath, mv8 = _sc_gather(payload, tab, sen3)
        s1 = _edge(gath, mv8, feat, wp)
        agg1 = jax.ops.segment_sum(s1, receivers, num_segments=N_NODES)
        if t == 0:
            cnt1 = agg1
        ns, nv = _update(agg1, cnt1, ns, nv, wp)

    return _readout(ns, nv, cnt1, params)


# speedup vs baseline: 10.7934x; 1.0004x over previous
import functools
import jax
import jax.numpy as jnp
from jax import lax
from jax.experimental import pallas as pl
from jax.experimental.pallas import tpu as pltpu
from jax.experimental.pallas import tpu_sc as plsc

N_NODES = 10000
N_EDGES = 320000
F = 128
NB = 4
RC = 1.0
M0 = 64
M1 = 21
STEPS = 3

BE = 4000
BN = 2000
SW = 152
SQRT3 = 1.7320508075688772

CHUNK = 80
NW = 32
E2 = N_EDGES // CHUNK
PER_W = E2 // NW
TABW = 40960

_SC_PARAMS = pltpu.CompilerParams(needs_layout_passes=False)


def _sc_mesh():
    return plsc.VectorSubcoreMesh(core_axis_name="c", subcore_axis_name="s")


def _sc_gather(payload, tab, idx3):
    @functools.partial(
        pl.kernel, mesh=_sc_mesh(),
        out_type=[jax.ShapeDtypeStruct((N_EDGES, F), jnp.float32),
                  jax.ShapeDtypeStruct((N_EDGES, 8), jnp.float32)],
        scratch_types=[pltpu.VMEM((PER_W, CHUNK), jnp.int32),
                       pltpu.VMEM((CHUNK, F), jnp.float32),
                       pltpu.VMEM((TABW,), jnp.float32),
                       pltpu.VMEM((CHUNK, 8), jnp.float32),
                       pltpu.SemaphoreType.DMA],
        compiler_params=_SC_PARAMS)
    def k(p_hbm, tab_hbm, idx_hbm, out_hbm, mv_hbm,
          idx_v, rows_v, tab_v, mv_v, sem):
        wid = lax.axis_index("s") * 2 + lax.axis_index("c")
        base = wid * PER_W
        pltpu.sync_copy(tab_hbm, tab_v)
        pltpu.sync_copy(idx_hbm.at[wid], idx_v)
        lane = lax.iota(jnp.int32, 16)

        @pl.loop(0, PER_W)
        def _chunk(j):
            pltpu.async_copy(p_hbm.at[idx_v.at[j]], rows_v, sem).wait()
            for t in range(CHUNK // 16):
                i16 = idx_v[j, pl.ds(t * 16, 16)]
                for c in range(3):
                    vals = plsc.load_gather(tab_v, [i16 * 4 + c])
                    plsc.store_scatter(
                        mv_v, [lane + t * 16, jnp.full((16,), c, jnp.int32)],
                        vals)
            pltpu.sync_copy(rows_v, out_hbm.at[pl.ds((base + j) * CHUNK, CHUNK)])
            pltpu.sync_copy(mv_v, mv_hbm.at[pl.ds((base + j) * CHUNK, CHUNK)])

    return k(payload, tab, idx3)


def _sc_edge_vec(tab, sidx3, ridx3):
    @functools.partial(
        pl.kernel, mesh=_sc_mesh(),
        out_type=jax.ShapeDtypeStruct((N_EDGES, 8), jnp.float32),
        scratch_types=[pltpu.VMEM((PER_W, CHUNK), jnp.int32),
                       pltpu.VMEM((PER_W, CHUNK), jnp.int32),
                       pltpu.VMEM((TABW,), jnp.float32),
                       pltpu.VMEM((CHUNK, 8), jnp.float32),
                       pltpu.SemaphoreType.DMA],
        compiler_params=_SC_PARAMS)
    def k(tab_hbm, s_hbm, r_hbm, out_hbm, sidx_v, ridx_v, tab_v, r8_v, sem):
        wid = lax.axis_index("s") * 2 + lax.axis_index("c")
        base = wid * PER_W
        pltpu.sync_copy(tab_hbm, tab_v)
        pltpu.sync_copy(s_hbm.at[wid], sidx_v)
        pltpu.sync_copy(r_hbm.at[wid], ridx_v)
        lane = lax.iota(jnp.int32, 16)

        @pl.loop(0, PER_W)
        def _chunk(j):
            for t in range(CHUNK // 16):
                s16 = sidx_v[j, pl.ds(t * 16, 16)]
                r16 = ridx_v[j, pl.ds(t * 16, 16)]
                for c in range(3):
                    vs = plsc.load_gather(tab_v, [s16 * 4 + c])
                    vr = plsc.load_gather(tab_v, [r16 * 4 + c])
                    plsc.store_scatter(
                        r8_v, [lane + t * 16, jnp.full((16,), c, jnp.int32)],
                        vs - vr)
            pltpu.sync_copy(r8_v, out_hbm.at[pl.ds((base + j) * CHUNK, CHUNK)])

    return k(tab, sidx3, ridx3)


def _whole(shape):
    nd = len(shape)
    return pl.BlockSpec(shape, lambda i, _nd=nd: (0,) * _nd)


def _rows(bs, cols):
    return pl.BlockSpec((bs, cols), lambda i: (i, 0))


_HI = jax.lax.Precision.HIGHEST


def _dot(a, b):
    return jnp.dot(a, b, preferred_element_type=jnp.float32, precision=_HI)


def _doth(a, b):
    return jnp.dot(a, b, preferred_element_type=jnp.float32,
                   precision=_HI)


def _phase0_body(r8_ref, feat_ref):
    r = r8_ref[:, 0:3]
    d = jnp.sqrt(jnp.sum(r * r, axis=1, keepdims=True))
    u = r / (d + 1e-9)
    a1 = SQRT3 * u
    na = jnp.sum(a1 * a1, axis=1, keepdims=True)
    x = jnp.maximum(d, 1e-6)
    coef = jnp.sqrt(2.0 / RC) / x
    rb = jnp.concatenate(
        [coef * jnp.sin((k * jnp.pi / RC) * x) for k in range(1, NB + 1)],
        axis=1)
    feat_ref[...] = jnp.concatenate([a1, na, rb], axis=1)


def _phase0(r8):
    return pl.pallas_call(
        _phase0_body,
        grid=(N_EDGES // BE,),
        in_specs=[_rows(BE, 8)],
        out_specs=_rows(BE, 8),
        out_shape=jax.ShapeDtypeStruct((N_EDGES, 8), jnp.float32),
    )(r8)


def _prep_body(ns_ref, wes_ref, p_ref):
    p_ref[...] = _dot(ns_ref[...], wes_ref[...])


def _prep(ns, wes):
    return pl.pallas_call(
        _prep_body,
        grid=(N_NODES // BN,),
        in_specs=[_rows(BN, F), _whole((F, F))],
        out_specs=_rows(BN, F),
        out_shape=jax.ShapeDtypeStruct((N_NODES, F), jnp.float32),
    )(ns, wes)


def _edge_body(g_ref, mv_ref, feat_ref, r0_ref, r1_ref, r2_ref, w3m_ref,
               w3x_ref, wmsa_ref, wmsb_ref, wmva_ref, wvrows_ref, wev_ref,
               s1_ref):
    g = g_ref[...]
    mv = wev_ref[0, 0] * mv_ref[:, 0:3]
    a1 = feat_ref[:, 0:3]
    na = feat_ref[:, 3:4]
    rb = feat_ref[:, 4:8]
    h = jax.nn.gelu(_doth(rb, r0_ref[...]))
    h = jax.nn.gelu(_doth(h, r1_ref[...]))
    h = jax.nn.gelu(_doth(h, r2_ref[...]))
    wm = _doth(h, w3m_ref[...])
    wx = _doth(h, w3x_ref[...])
    mva1 = jnp.sum(mv * a1, axis=1, keepdims=True)
    t3 = jnp.concatenate([wx[:, 0:1], wx[:, 1:2] * mva1,
                          wx[:, 2:3] * na], axis=1)
    es = _doth(wm[:, 0:128] * g, wmsa_ref[...]) + _doth(t3, wmsb_ref[...])
    q = (_doth(wm[:, 128:256] * g, wmva_ref[...])
         + wx[:, 3:4] * wvrows_ref[0:1, :]
         + wx[:, 5:6] * wvrows_ref[2:3, :])
    pc = wx[:, 4:5] * wvrows_ref[1:2, :]
    evs = [a1[:, i:i + 1] * q + mv[:, i:i + 1] * pc for i in range(3)]
    ones = jnp.ones((g.shape[0], 1), jnp.float32)
    s1_ref[...] = jnp.concatenate(
        [es, evs[0], evs[1], evs[2], ones, a1], axis=1)


def _edge(gath, mv8, feat, wp):
    return pl.pallas_call(
        _edge_body,
        grid=(N_EDGES // BE,),
        in_specs=[_rows(BE, F), _rows(BE, 8), _rows(BE, 8),
                  _whole((NB, 128)), _whole((128, 128)), _whole((128, 128)),
                  _whole((128, 256)), _whole((128, 6)),
                  _whole((128, 85)), _whole((3, 85)),
                  _whole((128, 21)), _whole((3, 21)), _whole((1, 1))],
        out_specs=_rows(BE, SW),
        out_shape=jax.ShapeDtypeStruct((N_EDGES, SW), jnp.float32),
    )(gath, mv8, feat, wp['R0'], wp['R1'], wp['R2'], wp['W3m'], wp['W3x'],
      wp['WmsA'], wp['WmsB'], wp['WmvA'], wp['Wvrows'], wp['W_ev'])


def _upd_body(a1_ref, c_ref, ns_ref, nv_ref, wns_ref, wnv_ref,
              wbs_ref, wbv_ref, ns_out, nv_out):
    inv_sqE = 1.0 / jnp.sqrt(float(N_EDGES))
    scale = inv_sqE / jnp.maximum(c_ref[:, 148:149], 1.0)
    a1 = a1_ref[...]
    lin_s = a1[:, 0:85] * scale + _dot(ns_ref[...], wns_ref[...])
    gates = jax.nn.sigmoid(lin_s[:, 0:M1])
    scal = jax.nn.gelu(lin_s[:, M1:85])
    ns_out[...] = _dot(scal, wbs_ref[...])
    gw = gates * wbv_ref[0:1, :]
    evs = [a1[:, 85:106], a1[:, 106:127], a1[:, 127:148]]
    cols = []
    for i in range(3):
        lin_v = evs[i] * scale + nv_ref[:, i:i + 1] * wnv_ref[0:1, :]
        cols.append(jnp.sum(gw * lin_v, axis=1, keepdims=True))
    cols.append(jnp.zeros_like(cols[0]))
    nv_out[...] = jnp.concatenate(cols, axis=1)


def _update(agg1, cnt_src, ns, nv4, wp):
    return pl.pallas_call(
        _upd_body,
        grid=(N_NODES // BN,),
        in_specs=[_rows(BN, SW), _rows(BN, SW),
                  _rows(BN, F), _rows(BN, 4),
                  _whole((F, 85)), _whole((1, M1)), _whole((M0, F)),
                  _whole((1, M1))],
        out_specs=[_rows(BN, F), _rows(BN, 4)],
        out_shape=[jax.ShapeDtypeStruct((N_NODES, F), jnp.float32),
                   jax.ShapeDtypeStruct((N_NODES, 4), jnp.float32)],
    )(agg1, cnt_src, ns, nv4,
      wp['Wn_s'], wp['Wnv_row'], wp['Wb_s'], wp['Wbv_row'])


def _readout_body(ns_ref, nv_ref, c1_ref, wpa_ref, wpb_ref,
                  w0_ref, b0_ref, w1_ref, b1_ref, w2_ref, b2_ref,
                  w3_ref, b3_ref, out_ref, acc1, acc2):
    i = pl.program_id(0)
    nblk = pl.num_programs(0)

    @pl.when(i == 0)
    def _init():
        acc1[...] = jnp.zeros_like(acc1)
        acc2[0] = 0.0

    attr_s = c1_ref[:, 148:149] * (1.0 / N_EDGES)
    attr_v = c1_ref[:, 149:152] * (1.0 / N_EDGES)
    acc1[...] += jax.lax.dot_general(
        attr_s, ns_ref[...], (((0,), (0,)), ((), ())),
        preferred_element_type=jnp.float32, precision=_HI)
    u = jnp.sum(nv_ref[:, 0:3] * attr_v)
    acc2[0] += u

    @pl.when(i == nblk - 1)
    def _fin():
        pre = (_dot(acc1[...] * (1.0 / N_NODES), wpa_ref[...])
               + (acc2[0] * (1.0 / N_NODES)) * wpb_ref[...])
        x = jax.nn.gelu(_dot(pre, w0_ref[...]) + b0_ref[...])
        x = jax.nn.gelu(_dot(x, w1_ref[...]) + b1_ref[...])
        x = jax.nn.gelu(_dot(x, w2_ref[...]) + b2_ref[...])
        x = _dot(x, w3_ref[...]) + b3_ref[...]
        out_ref[...] = x


def _readout(ns, nv4, cnt1, params):
    mlp = params['mlp']
    wpa = params['W_pre'][:F]
    wpb = params['W_pre'][F:F + 1]
    ops = [ns, nv4, cnt1, wpa, wpb,
           mlp[0]['W'], mlp[0]['b'][None, :], mlp[1]['W'], mlp[1]['b'][None, :],
           mlp[2]['W'], mlp[2]['b'][None, :], mlp[3]['W'], mlp[3]['b'][None, :]]
    in_specs = [_rows(BN, F), _rows(BN, 4), _rows(BN, SW),
                _whole((F, F)), _whole((1, F))]
    for k in range(4):
        in_specs.append(_whole(tuple(mlp[k]['W'].shape)))
        in_specs.append(_whole((1,) + tuple(mlp[k]['b'].shape)))
    out = pl.pallas_call(
        _readout_body,
        grid=(N_NODES // BN,),
        in_specs=in_specs,
        out_specs=pl.BlockSpec((1, 1), lambda i: (0, 0)),
        out_shape=jax.ShapeDtypeStruct((1, 1), jnp.float32),
        scratch_shapes=[pltpu.VMEM((1, F), jnp.float32),
                        pltpu.SMEM((1,), jnp.float32)],
    )(*ops)
    return out[0]


def kernel(nodes_s, nodes_v, senders, receivers, params):
    senders = senders.astype(jnp.int32)
    receivers = receivers.astype(jnp.int32)
    sen3 = senders.reshape(NW, PER_W, CHUNK)
    rec3 = receivers.reshape(NW, PER_W, CHUNK)
    nv4 = jnp.pad(nodes_v[:, 0, :], ((0, 0), (0, 1)))

    r8 = _sc_edge_vec(jnp.pad(nv4.reshape(-1), (0, TABW - 4 * N_NODES)),
                      sen3, rec3)
    feat = _phase0(r8)

    ns = nodes_s
    nv = nv4
    cnt1 = None
    cnt2 = None
    for t in range(STEPS):
        p = params['s%d' % t]
        wp = {
            'R0': p['R0'], 'R1': p['R1'], 'R2': p['R2'],
            'W3m': jnp.concatenate([p['R3'][:, 0:128], p['R3'][:, 131:259]],
                                   axis=1),
            'W3x': jnp.concatenate([p['R3'][:, 128:131], p['R3'][:, 259:262]],
                                   axis=1),
            'WmsA': p['Wm_s'][:128], 'WmsB': p['Wm_s'][128:131],
            'WmvA': p['Wm_v'][:128], 'Wvrows': p['Wm_v'][128:131],
            'Wn_s': p['Wn_s'], 'Wnv_row': p['Wn_v'][0:1, :],
            'Wb_s': p['Wb_s'], 'Wbv_row': p['Wb_v'][:, 0][None, :],
            'W_ev': p['W_ev'],
        }
        payload = _prep(ns, p['W_es'])
        tab = jnp.pad(nv.reshape(-1), (0, TABW - 4 * N_NODES))
        gath, mv8 = _sc_gather(payload, tab, sen3)
        s1 = _edge(gath, mv8, feat, wp)
        agg1 = jax.ops.segment_sum(s1, receivers, num_segments=N_NODES)
        if t == 0:
            cnt1 = agg1
        ns, nv = _update(agg1, cnt1, ns, nv, wp)

    return _readout(ns, nv, cnt1, params)
